# baseline (speedup 1.0000x reference)
import functools

import numpy as np

import jax
import jax.numpy as jnp
from jax import lax
from jax.experimental import pallas as pl
from jax.experimental.pallas import tpu as pltpu
from jax.experimental.pallas import tpu_sc as plsc

_B = 4096
_L = 50
_NW = 32
_ROWS_W = _B // _NW
_QROWS = _ROWS_W // 4
_OCTS = _QROWS // 8
_CHUNKS = (8 * _L) // 16

_BASES = (0, 22, 26, 30)
_SCALED = (True, False, False, True)


def _consts():
    jj = np.arange(8 * _L)
    srcr = (jj // _L).astype(np.int32)
    srcl = (jj % _L).astype(np.int32)
    dstc = (2 * (jj % _L)).astype(np.int32)
    return jnp.asarray(srcr), jnp.asarray(srcl), jnp.asarray(dstc)


_mesh = plsc.VectorSubcoreMesh(core_axis_name="c", subcore_axis_name="s")


@functools.partial(
    pl.kernel,
    out_type=jax.ShapeDtypeStruct((_B, _L * 8), jnp.float32),
    name="lambda_signature_lookup",
    mesh=_mesh,
    compiler_params=pltpu.CompilerParams(needs_layout_passes=False),
    scratch_types=[
        [pltpu.VMEM((_L, _ROWS_W), jnp.float32) for _ in range(4)],
        [pltpu.VMEM((_QROWS, _L * 8), jnp.float32) for _ in range(2)],
        pltpu.VMEM((64,), jnp.float32),
        pltpu.VMEM((3 * 8 * _L,), jnp.int32),
        pltpu.SemaphoreType.DMA,
        [pltpu.SemaphoreType.DMA for _ in range(2)],
    ],
)
def _sc_lookup(sigsT_hbm, tab_hbm, idx_hbm, out_hbm,
               in_bufs, out_bufs, tab_v, idx_v, in_sem, out_sems):
    wid = lax.axis_index("s") * 2 + lax.axis_index("c")
    base_row = wid * _ROWS_W

    def in_copy(f):
        return pltpu.make_async_copy(
            sigsT_hbm.at[f, :, pl.ds(base_row, _ROWS_W)], in_bufs[f], in_sem)

    for f in range(4):
        in_copy(f).start()
    pltpu.sync_copy(tab_hbm, tab_v)
    pltpu.sync_copy(idx_hbm, idx_v)

    def out_copy(q):
        rows = pl.ds(base_row + q * _QROWS, _QROWS)
        return pltpu.make_async_copy(out_bufs[q % 2], out_hbm.at[rows],
                                     out_sems[q % 2])

    for f in range(4):
        in_copy(f).wait()

    for q in range(4):
        if q >= 2:
            out_copy(q - 2).wait()

        out_v = out_bufs[q % 2]
        for f in range(4):
            sv = in_bufs[f]

            @plsc.parallel_loop(0, _CHUNKS, 1, unroll=4)
            def k_body(k, sv=sv, base=_BASES[f], scaled=_SCALED[f],
                       fcol=f * 100, out_v=out_v, qoff=q * _QROWS):
                k16 = k * 16
                sr = idx_v[pl.ds(k16, 16)]
                sl = idx_v[pl.ds(400 + k16, 16)]
                c0 = idx_v[pl.ds(800 + k16, 16)] + fcol

                for o in range(_OCTS):
                    ro = sr + o * 8
                    s = plsc.load_gather(sv, [sl, ro + qoff])
                    if scaled:
                        s = s * jnp.float32(10.0)
                    t = s.astype(jnp.int32)
                    idx = t + t + base
                    v0 = plsc.load_gather(tab_v, [idx])
                    v1 = plsc.load_gather(tab_v, [idx + 1])
                    plsc.store_scatter(out_v, [ro, c0], v0)
                    plsc.store_scatter(out_v, [ro, c0 + 1], v1)

        out_copy(q).start()

    out_copy(2).wait()
    out_copy(3).wait()


def kernel(sigs, frac_applicable_embed, bool_true_embed, bool_false_embed, frac_tf_embed):
    B, L, _ = sigs.shape
    tab = jnp.concatenate([
        frac_applicable_embed.reshape(-1),
        bool_true_embed.reshape(-1),
        bool_false_embed.reshape(-1),
        frac_tf_embed.reshape(-1),
    ])
    tab = jnp.pad(tab, (0, 64 - tab.shape[0]))
    srcr, srcl, dstc = _consts()
    idx = jnp.concatenate([srcr, srcl, dstc])
    sigsT = jnp.transpose(sigs, (2, 1, 0))
    return _sc_lookup(sigsT, tab, idx)


# speedup vs baseline: 1.1002x; 1.1002x over previous
import functools

import numpy as np

import jax
import jax.numpy as jnp
from jax import lax
from jax.experimental import pallas as pl
from jax.experimental.pallas import tpu as pltpu
from jax.experimental.pallas import tpu_sc as plsc

_B = 4096
_L = 50
_NW = 32
_ROWS_W = _B // _NW
_QROWS = _ROWS_W // 4
_OCTS = _QROWS // 8
_CHUNKS = (8 * _L) // 16

_BASES = (0, 22, 26, 30)
_SCALED = (True, False, False, True)


def _consts():
    jj = np.arange(8 * _L)
    srcr = (jj // _L).astype(np.int32)
    srcl = (jj % _L).astype(np.int32)
    dstc = (2 * (jj % _L)).astype(np.int32)
    return jnp.asarray(srcr), jnp.asarray(srcl), jnp.asarray(dstc)


_mesh = plsc.VectorSubcoreMesh(core_axis_name="c", subcore_axis_name="s")


@functools.partial(
    pl.kernel,
    out_type=jax.ShapeDtypeStruct((_B, _L * 8), jnp.float32),
    name="lambda_signature_lookup",
    mesh=_mesh,
    compiler_params=pltpu.CompilerParams(needs_layout_passes=False),
    scratch_types=[
        [pltpu.VMEM((_L, _ROWS_W), jnp.float32) for _ in range(4)],
        [pltpu.VMEM((_QROWS, _L * 8), jnp.float32) for _ in range(2)],
        pltpu.VMEM((64,), jnp.float32),
        pltpu.VMEM((3 * 8 * _L,), jnp.int32),
        pltpu.SemaphoreType.DMA,
        [pltpu.SemaphoreType.DMA for _ in range(2)],
    ],
)
def _sc_lookup(sigsT_hbm, tab_hbm, idx_hbm, out_hbm,
               in_bufs, out_bufs, tab_v, idx_v, in_sem, out_sems):
    wid = lax.axis_index("s") * 2 + lax.axis_index("c")
    base_row = wid * _ROWS_W

    def in_copy(f):
        return pltpu.make_async_copy(
            sigsT_hbm.at[f, :, pl.ds(base_row, _ROWS_W)], in_bufs[f], in_sem)

    for f in range(4):
        in_copy(f).start()
    pltpu.sync_copy(tab_hbm, tab_v)
    pltpu.sync_copy(idx_hbm, idx_v)

    def out_copy(q):
        rows = pl.ds(base_row + q * _QROWS, _QROWS)
        return pltpu.make_async_copy(out_bufs[q % 2], out_hbm.at[rows],
                                     out_sems[q % 2])

    for f in range(4):
        in_copy(f).wait()

    for q in range(4):
        if q >= 2:
            out_copy(q - 2).wait()

        out_v = out_bufs[q % 2]
        for f in range(4):
            sv = in_bufs[f]

            def k_body(k, _, sv=sv, base=_BASES[f], scaled=_SCALED[f],
                       fcol=f * 100, out_v=out_v, qoff=q * _QROWS):
                k16 = k * 16
                sr = idx_v[pl.ds(k16, 16)]
                sl = idx_v[pl.ds(400 + k16, 16)]
                c0 = idx_v[pl.ds(800 + k16, 16)] + fcol

                @plsc.parallel_loop(0, _OCTS, 1, unroll=4)
                def o_body(o):
                    ro = sr + o * 8
                    s = plsc.load_gather(sv, [sl, ro + qoff])
                    if scaled:
                        s = s * jnp.float32(10.0)
                    t = s.astype(jnp.int32)
                    idx = t + t + base
                    v0 = plsc.load_gather(tab_v, [idx])
                    v1 = plsc.load_gather(tab_v, [idx + 1])
                    plsc.store_scatter(out_v, [ro, c0], v0)
                    plsc.store_scatter(out_v, [ro, c0 + 1], v1)

                return 0

            lax.fori_loop(0, _CHUNKS, k_body, 0)

        out_copy(q).start()

    out_copy(2).wait()
    out_copy(3).wait()


def kernel(sigs, frac_applicable_embed, bool_true_embed, bool_false_embed, frac_tf_embed):
    B, L, _ = sigs.shape
    tab = jnp.concatenate([
        frac_applicable_embed.reshape(-1),
        bool_true_embed.reshape(-1),
        bool_false_embed.reshape(-1),
        frac_tf_embed.reshape(-1),
    ])
    tab = jnp.pad(tab, (0, 64 - tab.shape[0]))
    srcr, srcl, dstc = _consts()
    idx = jnp.concatenate([srcr, srcl, dstc])
    sigsT = jnp.transpose(sigs, (2, 1, 0))
    return _sc_lookup(sigsT, tab, idx)
